# Initial kernel scaffold; baseline (speedup 1.0000x reference)
import jax
import jax.numpy as jnp
from jax.experimental import pallas as pl


def kernel(edge_emb, edge_index, distance_vec, lattice, batch, rbf, W1, W2, W_rbf, W_out):
    raise NotImplementedError("write your pallas kernel here")


# speedup vs baseline: 33.2850x; 33.2850x over previous
import jax
import jax.numpy as jnp
from jax.experimental import pallas as pl
from jax.experimental.pallas import tpu as pltpu

_SCALE = 1.0 / 0.6


def _pick_block(e: int) -> int:
    for cand in (3200, 2560, 2000, 1600, 1280, 800, 640, 400, 320, 160, 80, 40, 8):
        if e % cand == 0:
            return cand
    return e


def _fused_kernel(src_ref, emb_ref, rbf_ref, dvec_ref, batch_ref,
                  w1_ref, w2_ref, wrbf_ref, wout_ref,
                  out_ref, acc_ref, starts_ref):
    i = pl.program_id(0)
    nb = pl.num_programs(0)
    bsz = out_ref.shape[0]

    @pl.when(i == 0)
    def _init():
        acc_ref[:] = jnp.zeros_like(acc_ref)
        b_ids = jax.lax.broadcasted_iota(jnp.int32, (bsz, 1), 0)
        lt = (batch_ref[:] < b_ids).astype(jnp.int32)
        starts_ref[0, :] = jnp.sum(lt, axis=1)

    h = jnp.dot(emb_ref[:], w1_ref[:], preferred_element_type=jnp.float32)
    h = jax.nn.silu(h) * _SCALE
    h = jnp.dot(h, w2_ref[:], preferred_element_type=jnp.float32)
    r = jnp.dot(rbf_ref[:], wrbf_ref[:], preferred_element_type=jnp.float32)
    s = jnp.sum(h * r * wout_ref[:], axis=1, keepdims=True)

    d = dvec_ref[:]
    norm = jnp.sqrt(jnp.sum(d * d, axis=1, keepdims=True))
    w = s / norm

    outer = jnp.concatenate(
        [d * d[:, 0:1], d * d[:, 1:2], d * d[:, 2:3]], axis=1)
    m = jnp.concatenate([outer * w, jnp.ones_like(s)], axis=1)

    src = src_ref[:]
    ge = (src >= starts_ref[:]).astype(jnp.int32)
    bidx = jnp.sum(ge, axis=1, keepdims=True) - 1
    lanes = jax.lax.broadcasted_iota(jnp.int32, (1, bsz), 1)
    onehot = (bidx == lanes).astype(jnp.float32)
    acc_ref[:] += jax.lax.dot_general(
        onehot, m, dimension_numbers=(((0,), (0,)), ((), ())),
        preferred_element_type=jnp.float32)

    @pl.when(i == nb - 1)
    def _fin():
        acc = acc_ref[:]
        cnt = acc[:, 9:10]
        lat = acc[:, 0:9]
        out_ref[:] = jnp.where(cnt > 0, lat / cnt, 0.0)


def kernel(edge_emb, edge_index, distance_vec, lattice, batch, rbf, W1, W2, W_rbf, W_out):
    e, d_dim = edge_emb.shape
    n = batch.shape[0]
    b = lattice.shape[0]
    r_dim = rbf.shape[1]
    blk = _pick_block(e)
    grid = (e // blk,)

    src = edge_index[0].astype(jnp.int32).reshape(e, 1)
    batch2d = batch.astype(jnp.int32).reshape(1, n)
    wout_row = W_out.astype(jnp.float32).reshape(1, d_dim)

    out = pl.pallas_call(
        _fused_kernel,
        grid=grid,
        in_specs=[
            pl.BlockSpec((blk, 1), lambda i: (i, 0)),
            pl.BlockSpec((blk, d_dim), lambda i: (i, 0)),
            pl.BlockSpec((blk, r_dim), lambda i: (i, 0)),
            pl.BlockSpec((blk, 3), lambda i: (i, 0)),
            pl.BlockSpec((1, n), lambda i: (0, 0)),
            pl.BlockSpec((d_dim, d_dim), lambda i: (0, 0)),
            pl.BlockSpec((d_dim, d_dim), lambda i: (0, 0)),
            pl.BlockSpec((r_dim, d_dim), lambda i: (0, 0)),
            pl.BlockSpec((1, d_dim), lambda i: (0, 0)),
        ],
        out_specs=pl.BlockSpec((b, 9), lambda i: (0, 0)),
        out_shape=jax.ShapeDtypeStruct((b, 9), jnp.float32),
        scratch_shapes=[
            pltpu.VMEM((b, 10), jnp.float32),
            pltpu.VMEM((1, b), jnp.int32),
        ],
    )(src, edge_emb, rbf, distance_vec, batch2d, W1, W2, W_rbf, wout_row)

    lat = out.reshape(b, 3, 3)
    return 0.5 * (lat + jnp.swapaxes(lat, 1, 2))


# speedup vs baseline: 37.2410x; 1.1189x over previous
import jax
import jax.numpy as jnp
from jax.experimental import pallas as pl
from jax.experimental.pallas import tpu as pltpu

_SCALE = 1.0 / 0.6


def _pick_block(e: int) -> int:
    for cand in (3200, 2560, 2000, 1600, 1280, 800, 640, 400, 320, 160, 80, 40, 8):
        if e % cand == 0:
            return cand
    return e


def _fused_kernel(src_ref, emb_ref, rbf_ref, dvec_ref, batch_ref,
                  w1_ref, w2_ref, wrbf_ref, wout_ref,
                  out_ref, acc_ref, starts_ref):
    i = pl.program_id(0)
    nb = pl.num_programs(0)
    bsz = out_ref.shape[0]

    @pl.when(i == 0)
    def _init():
        acc_ref[:] = jnp.zeros_like(acc_ref)
        b_ids = jax.lax.broadcasted_iota(jnp.int32, (bsz, 1), 0)
        lt = (batch_ref[:] < b_ids).astype(jnp.int32)
        starts_ref[0, :] = jnp.sum(lt, axis=1)

    h = jnp.dot(emb_ref[:], w1_ref[:], preferred_element_type=jnp.float32)
    h = jax.nn.silu(h) * _SCALE
    h = jnp.dot(h, w2_ref[:], preferred_element_type=jnp.float32)
    r = jnp.dot(rbf_ref[:], wrbf_ref[:], preferred_element_type=jnp.float32)
    s = jnp.dot(h * r, wout_ref[:], preferred_element_type=jnp.float32)

    d = dvec_ref[:]
    nsq = jnp.sum(d * d, axis=1, keepdims=True)
    w = s / jnp.sqrt(nsq)
    dw = d * w

    ge = (src_ref[:] >= starts_ref[:]).astype(jnp.float32)
    bidx = jnp.dot(ge, jnp.ones((bsz, 1), jnp.float32),
                   preferred_element_type=jnp.float32) - 1.0
    lanes = jax.lax.broadcasted_iota(jnp.int32, (1, bsz), 1).astype(jnp.float32)
    onehot = (bidx == lanes).astype(jnp.float32)

    d4 = jnp.concatenate([d, jnp.ones_like(s)], axis=1)
    dn = (((0,), (0,)), ((), ()))
    for k in range(3):
        acc_ref[bsz * k:bsz * (k + 1), :] += jax.lax.dot_general(
            onehot * dw[:, k:k + 1], d4, dimension_numbers=dn,
            preferred_element_type=jnp.float32)
    acc_ref[bsz * 3:bsz * 4, :] += jax.lax.dot_general(
        onehot, d4, dimension_numbers=dn,
        preferred_element_type=jnp.float32)

    @pl.when(i == nb - 1)
    def _fin():
        a = acc_ref[:]
        cnt = a[bsz * 3:bsz * 4, 3:4]
        lat = jnp.concatenate(
            [a[0:bsz, 0:3], a[bsz:2 * bsz, 0:3], a[2 * bsz:3 * bsz, 0:3]],
            axis=1)
        out_ref[:] = jnp.where(cnt > 0, lat / cnt, 0.0)


def kernel(edge_emb, edge_index, distance_vec, lattice, batch, rbf, W1, W2, W_rbf, W_out):
    e, d_dim = edge_emb.shape
    n = batch.shape[0]
    b = lattice.shape[0]
    r_dim = rbf.shape[1]
    blk = _pick_block(e)
    grid = (e // blk,)

    src = edge_index[0].astype(jnp.int32).reshape(e, 1)
    batch2d = batch.astype(jnp.int32).reshape(1, n)
    wout_col = W_out.astype(jnp.float32).reshape(d_dim, 1)

    out = pl.pallas_call(
        _fused_kernel,
        grid=grid,
        in_specs=[
            pl.BlockSpec((blk, 1), lambda i: (i, 0)),
            pl.BlockSpec((blk, d_dim), lambda i: (i, 0)),
            pl.BlockSpec((blk, r_dim), lambda i: (i, 0)),
            pl.BlockSpec((blk, 3), lambda i: (i, 0)),
            pl.BlockSpec((1, n), lambda i: (0, 0)),
            pl.BlockSpec((d_dim, d_dim), lambda i: (0, 0)),
            pl.BlockSpec((d_dim, d_dim), lambda i: (0, 0)),
            pl.BlockSpec((r_dim, d_dim), lambda i: (0, 0)),
            pl.BlockSpec((d_dim, 1), lambda i: (0, 0)),
        ],
        out_specs=pl.BlockSpec((b, 9), lambda i: (0, 0)),
        out_shape=jax.ShapeDtypeStruct((b, 9), jnp.float32),
        scratch_shapes=[
            pltpu.VMEM((4 * b, 4), jnp.float32),
            pltpu.VMEM((1, b), jnp.int32),
        ],
    )(src, edge_emb, rbf, distance_vec, batch2d, W1, W2, W_rbf, wout_col)

    lat = out.reshape(b, 3, 3)
    return 0.5 * (lat + jnp.swapaxes(lat, 1, 2))


# speedup vs baseline: 37.3751x; 1.0036x over previous
import jax
import jax.numpy as jnp
from jax.experimental import pallas as pl
from jax.experimental.pallas import tpu as pltpu

_SCALE = 1.0 / 0.6


def _pick_block(e: int) -> int:
    for cand in (3200, 2560, 2000, 1600, 1280, 800, 640, 400, 320, 160, 80, 40, 8):
        if e % cand == 0:
            return cand
    return e


def _fused_kernel(src_ref, emb_ref, rbf_ref, dvec_ref, batch_ref,
                  w1_ref, w2_ref, wrbf_ref, wout_ref,
                  out_ref, acc_ref, starts_ref, ends_ref):
    i = pl.program_id(0)
    nb = pl.num_programs(0)
    bsz = out_ref.shape[0]

    @pl.when(i == 0)
    def _init():
        acc_ref[:] = jnp.zeros_like(acc_ref)
        b_ids = jax.lax.broadcasted_iota(jnp.int32, (bsz, 1), 0)
        lt = (batch_ref[:] < b_ids).astype(jnp.int32)
        le = (batch_ref[:] <= b_ids).astype(jnp.int32)
        starts_ref[0, :] = jnp.sum(lt, axis=1)
        ends_ref[0, :] = jnp.sum(le, axis=1)

    h = jnp.dot(emb_ref[:], w1_ref[:], preferred_element_type=jnp.float32)
    h = jax.nn.silu(h)
    h = jnp.dot(h, w2_ref[:], preferred_element_type=jnp.float32)
    r = jnp.dot(rbf_ref[:], wrbf_ref[:], preferred_element_type=jnp.float32)
    s = jnp.dot(h * r, wout_ref[:], preferred_element_type=jnp.float32)

    d = dvec_ref[:]
    nsq = jnp.dot(d * d, jnp.ones((3, 1), jnp.float32),
                  preferred_element_type=jnp.float32)
    w = s * jax.lax.rsqrt(nsq)
    dw = d * w

    src = src_ref[:]
    onehot = ((src >= starts_ref[:]).astype(jnp.float32)
              - (src >= ends_ref[:]).astype(jnp.float32))

    d4 = jnp.concatenate([d, jnp.ones_like(s)], axis=1)
    dn = (((0,), (0,)), ((), ()))
    for k in range(3):
        acc_ref[bsz * k:bsz * (k + 1), :] += jax.lax.dot_general(
            onehot * dw[:, k:k + 1], d4, dimension_numbers=dn,
            preferred_element_type=jnp.float32)
    acc_ref[bsz * 3:bsz * 4, :] += jax.lax.dot_general(
        onehot, d4, dimension_numbers=dn,
        preferred_element_type=jnp.float32)

    @pl.when(i == nb - 1)
    def _fin():
        a = acc_ref[:]
        cnt = a[bsz * 3:bsz * 4, 3:4]
        lat = jnp.concatenate(
            [a[0:bsz, 0:3], a[bsz:2 * bsz, 0:3], a[2 * bsz:3 * bsz, 0:3]],
            axis=1)
        out_ref[:] = jnp.where(cnt > 0, lat / cnt, 0.0)


def kernel(edge_emb, edge_index, distance_vec, lattice, batch, rbf, W1, W2, W_rbf, W_out):
    e, d_dim = edge_emb.shape
    n = batch.shape[0]
    b = lattice.shape[0]
    r_dim = rbf.shape[1]
    blk = _pick_block(e)
    grid = (e // blk,)

    src = edge_index[0].astype(jnp.int32).reshape(e, 1)
    batch2d = batch.astype(jnp.int32).reshape(1, n)
    wout_col = W_out.astype(jnp.float32).reshape(d_dim, 1)
    w2_scaled = W2 * jnp.float32(_SCALE)

    out = pl.pallas_call(
        _fused_kernel,
        grid=grid,
        in_specs=[
            pl.BlockSpec((blk, 1), lambda i: (i, 0)),
            pl.BlockSpec((blk, d_dim), lambda i: (i, 0)),
            pl.BlockSpec((blk, r_dim), lambda i: (i, 0)),
            pl.BlockSpec((blk, 3), lambda i: (i, 0)),
            pl.BlockSpec((1, n), lambda i: (0, 0)),
            pl.BlockSpec((d_dim, d_dim), lambda i: (0, 0)),
            pl.BlockSpec((d_dim, d_dim), lambda i: (0, 0)),
            pl.BlockSpec((r_dim, d_dim), lambda i: (0, 0)),
            pl.BlockSpec((d_dim, 1), lambda i: (0, 0)),
        ],
        out_specs=pl.BlockSpec((b, 9), lambda i: (0, 0)),
        out_shape=jax.ShapeDtypeStruct((b, 9), jnp.float32),
        scratch_shapes=[
            pltpu.VMEM((4 * b, 4), jnp.float32),
            pltpu.VMEM((1, b), jnp.int32),
            pltpu.VMEM((1, b), jnp.int32),
        ],
    )(src, edge_emb, rbf, distance_vec, batch2d, W1, w2_scaled, W_rbf, wout_col)

    lat = out.reshape(b, 3, 3)
    return 0.5 * (lat + jnp.swapaxes(lat, 1, 2))
